# Initial kernel scaffold; baseline (speedup 1.0000x reference)
import jax
import jax.numpy as jnp
from jax.experimental import pallas as pl


def kernel(input, weight):
    raise NotImplementedError("write your pallas kernel here")


# speedup vs baseline: 1.3808x; 1.3808x over previous
import functools
import math

import jax
import jax.numpy as jnp
from jax import lax
from jax.experimental import pallas as pl
from jax.experimental.pallas import tpu as pltpu
from jax.experimental.pallas import tpu_sc as plsc

_K = 512
_D = 32
_B = 128
_T = 1024
_N = _B * _T
_BLK = 512
_GRID = _N // _BLK

_NC = 2
_NS = 16
_NW = _NC * _NS
_BPW = _N // _NW
_CH = 1024


def _tc_body(z_ref, wt_ref, idx_ref, emb_ref, ent_ref, cnt_ref):
    i = pl.program_id(0)
    z = z_ref[...]
    wt = wt_ref[...]
    sqr = jnp.sum(wt * wt, axis=0, keepdims=True)
    cov = jnp.dot(z, wt, preferred_element_type=jnp.float32)
    scores = sqr - 2.0 * cov
    minval = jnp.min(scores, axis=1)
    iota = lax.broadcasted_iota(jnp.int32, (_BLK, _K), 1)
    hit = scores <= minval[:, None]
    idx = jnp.min(jnp.where(hit, iota, _K), axis=1)
    idx_ref[...] = idx
    part = jnp.sum(minval) + jnp.sum(z * z)
    onehot = (idx[:, None] == lax.broadcasted_iota(jnp.int32, (1, _K), 1))
    cnt = jnp.sum(onehot.astype(jnp.float32), axis=0, keepdims=True)

    @pl.when(i == 0)
    def _init():
        emb_ref[0, 0] = part
        cnt_ref[...] = cnt

    @pl.when(i > 0)
    def _acc():
        emb_ref[0, 0] += part
        cnt_ref[...] += cnt

    @pl.when(i == _GRID - 1)
    def _fini():
        p = cnt_ref[...] * (1.0 / _N)
        ent_ref[0, 0] = -jnp.sum(jnp.where(p > 0, p * jnp.log(p), 0.0)) * (
            1.0 / math.log(2.0))


def _tc_call(z, wt):
    return pl.pallas_call(
        _tc_body,
        grid=(_GRID,),
        in_specs=[
            pl.BlockSpec((_BLK, _D), lambda i: (i, 0)),
            pl.BlockSpec((_D, _K), lambda i: (0, 0)),
        ],
        out_specs=[
            pl.BlockSpec((_BLK,), lambda i: (i,)),
            pl.BlockSpec((1, 1), lambda i: (0, 0), memory_space=pltpu.SMEM),
            pl.BlockSpec((1, 1), lambda i: (0, 0), memory_space=pltpu.SMEM),
        ],
        out_shape=[
            jax.ShapeDtypeStruct((_N,), jnp.int32),
            jax.ShapeDtypeStruct((1, 1), jnp.float32),
            jax.ShapeDtypeStruct((1, 1), jnp.float32),
        ],
        scratch_shapes=[pltpu.VMEM((1, _K), jnp.float32)],
    )(z, wt)


@functools.cache
def _make_sc_gather():
    mesh = plsc.VectorSubcoreMesh(core_axis_name="c", subcore_axis_name="s")
    n_chunks = _BPW // _CH

    @functools.partial(
        pl.kernel,
        mesh=mesh,
        out_type=jax.ShapeDtypeStruct((_N * _D,), jnp.float32),
        scratch_types=[
            pltpu.VMEM((_K * _D,), jnp.float32),
            pltpu.VMEM((_CH,), jnp.int32),
            pltpu.VMEM((_CH * _D,), jnp.float32),
        ],
    )
    def _sc_gather(w_hbm, idx_hbm, out_hbm, tab_v, idx_v, out_v):
        wid = lax.axis_index("s") * _NC + lax.axis_index("c")
        base = wid * _BPW
        pltpu.sync_copy(w_hbm, tab_v)
        for ch in range(n_chunks):
            tok0 = base + ch * _CH
            pltpu.sync_copy(idx_hbm.at[pl.ds(tok0, _CH)], idx_v)

            def body(g, carry):
                c_vec = idx_v[pl.ds(g * 16, 16)]
                for l in range(16):
                    b = c_vec[l] * _D
                    t = (g * 16 + l) * _D
                    out_v[pl.ds(t, 16)] = tab_v[pl.ds(b, 16)]
                    out_v[pl.ds(t + 16, 16)] = tab_v[pl.ds(b + 16, 16)]
                return carry

            lax.fori_loop(0, _CH // 16, body, 0)
            pltpu.sync_copy(out_v, out_hbm.at[pl.ds(tok0 * _D, _CH * _D)])

    return _sc_gather


def kernel(input, weight):
    z = input.reshape(_N, _D)
    idx, emb, ent = _tc_call(z, weight.T)
    x_hat = _make_sc_gather()(weight.reshape(_K * _D), idx)
    emb_s = emb[0, 0]
    return (
        x_hat.reshape(_B, _T, _D),
        idx.reshape(_B, _T),
        emb_s,
        emb_s,
        ent[0, 0],
    )


# speedup vs baseline: 1.6062x; 1.1633x over previous
import functools
import math

import jax
import jax.numpy as jnp
from jax import lax
from jax.experimental import pallas as pl
from jax.experimental.pallas import tpu as pltpu
from jax.experimental.pallas import tpu_sc as plsc

_K = 512
_D = 32
_B = 128
_T = 1024
_N = _B * _T
_BLK = 512
_GRID = _N // _BLK

_NC = 2
_NS = 16
_NW = _NC * _NS
_BPW = _N // _NW
_CH = 1024


def _tc_body(z_ref, wt_ref, idx_ref, emb_ref, ent_ref, cnt_ref):
    i = pl.program_id(0)
    z = z_ref[...]
    wt = wt_ref[...]
    sqr = jnp.sum(wt * wt, axis=0, keepdims=True)
    cov = jnp.dot(z, wt, preferred_element_type=jnp.float32)
    scores = sqr - 2.0 * cov
    minval = jnp.min(scores, axis=1)
    iota_f = lax.broadcasted_iota(jnp.int32, (_BLK, _K), 1).astype(jnp.float32)
    hit = scores <= minval[:, None]
    idx_f = jnp.min(jnp.where(hit, iota_f, float(_K)), axis=1)
    idx = idx_f.astype(jnp.int32)
    idx_ref[...] = idx
    part = jnp.sum(minval) + jnp.sum(z * z)
    onehot = (idx[:, None] == lax.broadcasted_iota(jnp.int32, (1, _K), 1))
    cnt = jnp.sum(onehot.astype(jnp.float32), axis=0, keepdims=True)

    @pl.when(i == 0)
    def _init():
        emb_ref[0, 0] = part
        cnt_ref[...] = cnt

    @pl.when(i > 0)
    def _acc():
        emb_ref[0, 0] += part
        cnt_ref[...] += cnt

    @pl.when(i == _GRID - 1)
    def _fini():
        p = cnt_ref[...] * (1.0 / _N)
        ent_ref[0, 0] = -jnp.sum(jnp.where(p > 0, p * jnp.log(p), 0.0)) * (
            1.0 / math.log(2.0))


def _tc_call(z, wt):
    return pl.pallas_call(
        _tc_body,
        grid=(_GRID,),
        in_specs=[
            pl.BlockSpec((_BLK, _D), lambda i: (i, 0)),
            pl.BlockSpec((_D, _K), lambda i: (0, 0)),
        ],
        out_specs=[
            pl.BlockSpec((_BLK,), lambda i: (i,)),
            pl.BlockSpec((1, 1), lambda i: (0, 0), memory_space=pltpu.SMEM),
            pl.BlockSpec((1, 1), lambda i: (0, 0), memory_space=pltpu.SMEM),
        ],
        out_shape=[
            jax.ShapeDtypeStruct((_N,), jnp.int32),
            jax.ShapeDtypeStruct((1, 1), jnp.float32),
            jax.ShapeDtypeStruct((1, 1), jnp.float32),
        ],
        scratch_shapes=[pltpu.VMEM((1, _K), jnp.float32)],
    )(z, wt)


@functools.cache
def _make_sc_gather():
    mesh = plsc.VectorSubcoreMesh(core_axis_name="c", subcore_axis_name="s")
    n_chunks = _BPW // _CH

    @functools.partial(
        pl.kernel,
        mesh=mesh,
        out_type=jax.ShapeDtypeStruct((_N * _D,), jnp.float32),
        scratch_types=[
            pltpu.VMEM((_K * _D,), jnp.float32),
            pltpu.VMEM((_CH,), jnp.int32),
            pltpu.VMEM((_CH * _D,), jnp.float32),
        ],
    )
    def _sc_gather(w_hbm, idx_hbm, out_hbm, tab_v, idx_v, out_v):
        wid = lax.axis_index("s") * _NC + lax.axis_index("c")
        base = wid * _BPW
        pltpu.sync_copy(w_hbm, tab_v)
        for ch in range(n_chunks):
            tok0 = base + ch * _CH
            pltpu.sync_copy(idx_hbm.at[pl.ds(tok0, _CH)], idx_v)

            def body(g, carry):
                c_vec = idx_v[pl.ds(g * 16, 16)]
                for l in range(16):
                    b = c_vec[l] * _D
                    t = (g * 16 + l) * _D
                    out_v[pl.ds(t, 16)] = tab_v[pl.ds(b, 16)]
                    out_v[pl.ds(t + 16, 16)] = tab_v[pl.ds(b + 16, 16)]
                return carry

            lax.fori_loop(0, _CH // 16, body, 0)
            pltpu.sync_copy(out_v, out_hbm.at[pl.ds(tok0 * _D, _CH * _D)])

    return _sc_gather


def kernel(input, weight):
    z = input.reshape(_N, _D)
    idx, emb, ent = _tc_call(z, weight.T)
    x_hat = _make_sc_gather()(weight.reshape(_K * _D), idx)
    emb_s = emb[0, 0]
    return (
        x_hat.reshape(_B, _T, _D),
        idx.reshape(_B, _T),
        emb_s,
        emb_s,
        ent[0, 0],
    )


# speedup vs baseline: 2.5892x; 1.6120x over previous
import functools
import math

import jax
import jax.numpy as jnp
from jax import lax
from jax.experimental import pallas as pl
from jax.experimental.pallas import tpu as pltpu
from jax.experimental.pallas import tpu_sc as plsc

_K = 512
_D = 32
_B = 128
_T = 1024
_N = _B * _T

_NC = 2
_NS = 16
_NW = _NC * _NS
_BPW = _N // _NW
_CH = 1024


def _tc_body(zt_ref, w_ref, idx_ref, emb_ref):
    b = pl.program_id(0)
    ztb = zt_ref[...]
    w = w_ref[...]
    sqr = jnp.sum(w * w, axis=1, keepdims=True)
    cov2 = jnp.dot(w + w, ztb, preferred_element_type=jnp.float32)
    scores = sqr - cov2
    minv = jnp.min(scores, axis=0)
    iota_c = lax.broadcasted_iota(jnp.int32, (_K, 1), 0).astype(jnp.float32)
    hit = scores <= minv[None, :]
    idx_f = jnp.min(jnp.where(hit, iota_c, float(_K)), axis=0)
    idx_ref[...] = idx_f.astype(jnp.int32)
    part = jnp.sum(minv) + jnp.sum(ztb * ztb)

    @pl.when(b == 0)
    def _init():
        emb_ref[0, 0] = part

    @pl.when(b > 0)
    def _acc():
        emb_ref[0, 0] += part


def _tc_call(zt, w):
    return pl.pallas_call(
        _tc_body,
        grid=(_B,),
        in_specs=[
            pl.BlockSpec((_D, _T), lambda b: (b, 0)),
            pl.BlockSpec((_K, _D), lambda b: (0, 0)),
        ],
        out_specs=[
            pl.BlockSpec((_T,), lambda b: (b,)),
            pl.BlockSpec((1, 1), lambda b: (0, 0), memory_space=pltpu.SMEM),
        ],
        out_shape=[
            jax.ShapeDtypeStruct((_N,), jnp.int32),
            jax.ShapeDtypeStruct((1, 1), jnp.float32),
        ],
    )(zt, w)


@functools.cache
def _make_sc_gather():
    mesh = plsc.VectorSubcoreMesh(core_axis_name="c", subcore_axis_name="s")
    n_chunks = _BPW // _CH

    @functools.partial(
        pl.kernel,
        mesh=mesh,
        out_type=[
            jax.ShapeDtypeStruct((_N * _D,), jnp.float32),
            jax.ShapeDtypeStruct((_NW * _K,), jnp.int32),
        ],
        scratch_types=[
            pltpu.VMEM((_K * _D,), jnp.float32),
            pltpu.VMEM((_CH,), jnp.int32),
            pltpu.VMEM((_CH * _D,), jnp.float32),
            pltpu.SMEM((_K,), jnp.int32),
            pltpu.VMEM((_K,), jnp.int32),
        ],
    )
    def _sc_gather(w_hbm, idx_hbm, out_hbm, hist_hbm,
                   tab_v, idx_v, out_v, hist_s, hist_v):
        wid = lax.axis_index("s") * _NC + lax.axis_index("c")
        base = wid * _BPW
        pltpu.sync_copy(w_hbm, tab_v)

        def zbody(k, carry):
            hist_s[k] = 0
            return carry

        lax.fori_loop(0, _K, zbody, 0)
        for ch in range(n_chunks):
            tok0 = base + ch * _CH
            pltpu.sync_copy(idx_hbm.at[pl.ds(tok0, _CH)], idx_v)

            def body(g, carry):
                c_vec = idx_v[pl.ds(g * 16, 16)]
                for l in range(16):
                    c = c_vec[l]
                    b = c * _D
                    t = (g * 16 + l) * _D
                    out_v[pl.ds(t, 16)] = tab_v[pl.ds(b, 16)]
                    out_v[pl.ds(t + 16, 16)] = tab_v[pl.ds(b + 16, 16)]
                    hist_s[c] = hist_s[c] + 1
                return carry

            lax.fori_loop(0, _CH // 16, body, 0)
            pltpu.sync_copy(out_v, out_hbm.at[pl.ds(tok0 * _D, _CH * _D)])
        lane = lax.iota(jnp.int32, 16)
        for k0 in range(_K // 16):
            vals = jnp.zeros((16,), jnp.int32)
            for l in range(16):
                vals = jnp.where(lane == l, hist_s[k0 * 16 + l], vals)
            hist_v[pl.ds(k0 * 16, 16)] = vals
        pltpu.sync_copy(hist_v, hist_hbm.at[pl.ds(wid * _K, _K)])

    return _sc_gather


def _ent_body(h_ref, ent_ref):
    cnt = jnp.sum(h_ref[...].astype(jnp.float32), axis=0, keepdims=True)
    p = cnt * (1.0 / _N)
    ent_ref[0, 0] = -jnp.sum(jnp.where(p > 0, p * jnp.log(p), 0.0)) * (
        1.0 / math.log(2.0))


def _ent_call(hist):
    return pl.pallas_call(
        _ent_body,
        grid=(1,),
        in_specs=[pl.BlockSpec((_NW, _K), lambda i: (0, 0))],
        out_specs=[pl.BlockSpec((1, 1), lambda i: (0, 0),
                                memory_space=pltpu.SMEM)],
        out_shape=[jax.ShapeDtypeStruct((1, 1), jnp.float32)],
    )(hist)


def kernel(input, weight):
    zt = input.transpose(0, 2, 1).reshape(_B * _D, _T)
    idx, emb = _tc_call(zt, weight)
    x_hat, hist = _make_sc_gather()(weight.reshape(_K * _D), idx)
    (ent,) = _ent_call(hist.reshape(_NW, _K))
    emb_s = emb[0, 0]
    return (
        x_hat.reshape(_B, _T, _D),
        idx.reshape(_B, _T),
        emb_s,
        emb_s,
        ent[0, 0],
    )


# speedup vs baseline: 2.6396x; 1.0194x over previous
import functools
import math

import jax
import jax.numpy as jnp
from jax import lax
from jax.experimental import pallas as pl
from jax.experimental.pallas import tpu as pltpu
from jax.experimental.pallas import tpu_sc as plsc

_K = 512
_D = 32
_B = 128
_T = 1024
_N = _B * _T

_NC = 2
_NS = 16
_NW = _NC * _NS
_BPW = _N // _NW
_CH = 1024


def _tc_body(zt_ref, w_ref, idx_ref, emb_ref):
    b = pl.program_id(0)
    ztb = zt_ref[...]
    w = w_ref[...]
    sqr = jnp.sum(w * w, axis=1, keepdims=True)
    cov2 = jnp.dot(w + w, ztb, preferred_element_type=jnp.float32)
    scores = sqr - cov2
    minv = jnp.min(scores, axis=0)
    iota_c = lax.broadcasted_iota(jnp.int32, (_K, 1), 0).astype(jnp.float32)
    hit = scores <= minv[None, :]
    idx_f = jnp.min(jnp.where(hit, iota_c, float(_K)), axis=0)
    idx_ref[...] = idx_f.astype(jnp.int32)
    part = jnp.sum(minv) + jnp.sum(ztb * ztb)

    @pl.when(b == 0)
    def _init():
        emb_ref[0, 0] = part

    @pl.when(b > 0)
    def _acc():
        emb_ref[0, 0] += part


def _tc_call(zt, w):
    return pl.pallas_call(
        _tc_body,
        grid=(_B,),
        in_specs=[
            pl.BlockSpec((_D, _T), lambda b: (b, 0)),
            pl.BlockSpec((_K, _D), lambda b: (0, 0)),
        ],
        out_specs=[
            pl.BlockSpec((_T,), lambda b: (b,)),
            pl.BlockSpec((1, 1), lambda b: (0, 0), memory_space=pltpu.SMEM),
        ],
        out_shape=[
            jax.ShapeDtypeStruct((_N,), jnp.int32),
            jax.ShapeDtypeStruct((1, 1), jnp.float32),
        ],
    )(zt, w)


@functools.cache
def _make_sc_gather():
    mesh = plsc.VectorSubcoreMesh(core_axis_name="c", subcore_axis_name="s")
    n_chunks = _BPW // _CH

    @functools.partial(
        pl.kernel,
        mesh=mesh,
        out_type=[
            jax.ShapeDtypeStruct((_N * _D,), jnp.float32),
            jax.ShapeDtypeStruct((_NW * _K,), jnp.int32),
        ],
        scratch_types=[
            pltpu.VMEM((_K * _D,), jnp.float32),
            pltpu.VMEM((2, _CH), jnp.int32),
            pltpu.VMEM((2, _CH * _D), jnp.float32),
            pltpu.SMEM((_K,), jnp.int32),
            pltpu.VMEM((_K,), jnp.int32),
            pltpu.SemaphoreType.DMA,
            pltpu.SemaphoreType.DMA,
            pltpu.SemaphoreType.DMA,
            pltpu.SemaphoreType.DMA,
        ],
    )
    def _sc_gather(w_hbm, idx_hbm, out_hbm, hist_hbm,
                   tab_v, idx_v, out_v, hist_s, hist_v,
                   sem_i0, sem_i1, sem_o0, sem_o1):
        wid = lax.axis_index("s") * _NC + lax.axis_index("c")
        base = wid * _BPW
        sem_i = (sem_i0, sem_i1)
        sem_o = (sem_o0, sem_o1)
        idx_cp = [None, None]
        out_cp = [None, None]
        idx_cp[0] = pltpu.async_copy(
            idx_hbm.at[pl.ds(base, _CH)], idx_v.at[0], sem_i[0])
        pltpu.sync_copy(w_hbm, tab_v)

        def zbody(k0, carry):
            for j in range(8):
                hist_s[k0 * 8 + j] = 0
            return carry

        lax.fori_loop(0, _K // 8, zbody, 0)
        for ch in range(n_chunks):
            cur = ch & 1
            nxt = 1 - cur
            if ch + 1 < n_chunks:
                idx_cp[nxt] = pltpu.async_copy(
                    idx_hbm.at[pl.ds(base + (ch + 1) * _CH, _CH)],
                    idx_v.at[nxt], sem_i[nxt])
            idx_cp[cur].wait()
            if out_cp[cur] is not None:
                out_cp[cur].wait()

            def body(g, carry):
                c_vec = idx_v[cur, pl.ds(g * 16, 16)]
                for l in range(16):
                    c = c_vec[l]
                    b = c * _D
                    t = (g * 16 + l) * _D
                    out_v[cur, pl.ds(t, 16)] = tab_v[pl.ds(b, 16)]
                    out_v[cur, pl.ds(t + 16, 16)] = tab_v[pl.ds(b + 16, 16)]
                    hist_s[c] = hist_s[c] + 1
                return carry

            lax.fori_loop(0, _CH // 16, body, 0)
            out_cp[cur] = pltpu.async_copy(
                out_v.at[cur],
                out_hbm.at[pl.ds((base + ch * _CH) * _D, _CH * _D)],
                sem_o[cur])
        for q in range(2):
            if out_cp[q] is not None:
                out_cp[q].wait()
        lane = lax.iota(jnp.int32, 16)
        for k0 in range(_K // 16):
            vals = jnp.zeros((16,), jnp.int32)
            for l in range(16):
                vals = jnp.where(lane == l, hist_s[k0 * 16 + l], vals)
            hist_v[pl.ds(k0 * 16, 16)] = vals
        pltpu.sync_copy(hist_v, hist_hbm.at[pl.ds(wid * _K, _K)])

    return _sc_gather


def _ent_body(h_ref, ent_ref):
    cnt = jnp.sum(h_ref[...].astype(jnp.float32), axis=0, keepdims=True)
    p = cnt * (1.0 / _N)
    ent_ref[0, 0] = -jnp.sum(jnp.where(p > 0, p * jnp.log(p), 0.0)) * (
        1.0 / math.log(2.0))


def _ent_call(hist):
    return pl.pallas_call(
        _ent_body,
        grid=(1,),
        in_specs=[pl.BlockSpec((_NW, _K), lambda i: (0, 0))],
        out_specs=[pl.BlockSpec((1, 1), lambda i: (0, 0),
                                memory_space=pltpu.SMEM)],
        out_shape=[jax.ShapeDtypeStruct((1, 1), jnp.float32)],
    )(hist)


def kernel(input, weight):
    zt = input.transpose(0, 2, 1).reshape(_B * _D, _T)
    idx, emb = _tc_call(zt, weight)
    x_hat, hist = _make_sc_gather()(weight.reshape(_K * _D), idx)
    (ent,) = _ent_call(hist.reshape(_NW, _K))
    emb_s = emb[0, 0]
    return (
        x_hat.reshape(_B, _T, _D),
        idx.reshape(_B, _T),
        emb_s,
        emb_s,
        ent[0, 0],
    )


# speedup vs baseline: 2.9050x; 1.1006x over previous
import functools
import math

import jax
import jax.numpy as jnp
from jax import lax
from jax.experimental import pallas as pl
from jax.experimental.pallas import tpu as pltpu
from jax.experimental.pallas import tpu_sc as plsc

_K = 512
_D = 32
_B = 128
_T = 1024
_N = _B * _T

_NC = 2
_NS = 16
_NW = _NC * _NS
_HB = _B // 2
_HN = _N // 2
_BPW = _HN // _NW
_CH = 1024


def _tc_body(zt_ref, w_ref, idx_ref, emb_ref):
    b = pl.program_id(0)
    ztb = zt_ref[...]
    w = w_ref[...]
    sqr = jnp.sum(w * w, axis=1, keepdims=True)
    cov2 = jnp.dot(w + w, ztb, preferred_element_type=jnp.float32)
    scores = sqr - cov2
    minv = jnp.min(scores, axis=0)
    iota_c = lax.broadcasted_iota(jnp.int32, (_K, 1), 0).astype(jnp.float32)
    hit = scores <= minv[None, :]
    idx_f = jnp.min(jnp.where(hit, iota_c, float(_K)), axis=0)
    idx_ref[...] = idx_f.astype(jnp.int32)
    part = jnp.sum(minv) + jnp.sum(ztb * ztb)

    @pl.when(b == 0)
    def _init():
        emb_ref[0, 0] = part

    @pl.when(b > 0)
    def _acc():
        emb_ref[0, 0] += part


def _tc_call(zt, w, half):
    off = half * _HB
    return pl.pallas_call(
        _tc_body,
        grid=(_HB,),
        in_specs=[
            pl.BlockSpec((_D, _T), lambda b: (b + off, 0)),
            pl.BlockSpec((_K, _D), lambda b: (0, 0)),
        ],
        out_specs=[
            pl.BlockSpec((_T,), lambda b: (b,)),
            pl.BlockSpec((1, 1), lambda b: (0, 0), memory_space=pltpu.SMEM),
        ],
        out_shape=[
            jax.ShapeDtypeStruct((_HN,), jnp.int32),
            jax.ShapeDtypeStruct((1, 1), jnp.float32),
        ],
    )(zt, w)


@functools.cache
def _make_sc_gather():
    mesh = plsc.VectorSubcoreMesh(core_axis_name="c", subcore_axis_name="s")
    n_chunks = _BPW // _CH

    @functools.partial(
        pl.kernel,
        mesh=mesh,
        out_type=[
            jax.ShapeDtypeStruct((_HN * _D,), jnp.float32),
            jax.ShapeDtypeStruct((_NW * _K,), jnp.int32),
        ],
        scratch_types=[
            pltpu.VMEM((_K * _D,), jnp.float32),
            pltpu.VMEM((2, _CH), jnp.int32),
            pltpu.VMEM((2, _CH * _D), jnp.float32),
            pltpu.SMEM((_K,), jnp.int32),
            pltpu.VMEM((_K,), jnp.int32),
            pltpu.SemaphoreType.DMA,
            pltpu.SemaphoreType.DMA,
            pltpu.SemaphoreType.DMA,
            pltpu.SemaphoreType.DMA,
        ],
    )
    def _sc_gather(w_hbm, idx_hbm, out_hbm, hist_hbm,
                   tab_v, idx_v, out_v, hist_s, hist_v,
                   sem_i0, sem_i1, sem_o0, sem_o1):
        wid = lax.axis_index("s") * _NC + lax.axis_index("c")
        base = wid * _BPW
        sem_i = (sem_i0, sem_i1)
        sem_o = (sem_o0, sem_o1)
        idx_cp = [None, None]
        out_cp = [None, None]
        idx_cp[0] = pltpu.async_copy(
            idx_hbm.at[pl.ds(base, _CH)], idx_v.at[0], sem_i[0])
        pltpu.sync_copy(w_hbm, tab_v)

        def zbody(k0, carry):
            for j in range(8):
                hist_s[k0 * 8 + j] = 0
            return carry

        lax.fori_loop(0, _K // 8, zbody, 0)
        for ch in range(n_chunks):
            cur = ch & 1
            nxt = 1 - cur
            if ch + 1 < n_chunks:
                idx_cp[nxt] = pltpu.async_copy(
                    idx_hbm.at[pl.ds(base + (ch + 1) * _CH, _CH)],
                    idx_v.at[nxt], sem_i[nxt])
            idx_cp[cur].wait()
            if out_cp[cur] is not None:
                out_cp[cur].wait()

            def body(g, carry):
                c_vec = idx_v[cur, pl.ds(g * 16, 16)]
                for l in range(16):
                    c = c_vec[l]
                    b = c * _D
                    t = (g * 16 + l) * _D
                    out_v[cur, pl.ds(t, 16)] = tab_v[pl.ds(b, 16)]
                    out_v[cur, pl.ds(t + 16, 16)] = tab_v[pl.ds(b + 16, 16)]
                    hist_s[c] = hist_s[c] + 1
                return carry

            lax.fori_loop(0, _CH // 16, body, 0)
            out_cp[cur] = pltpu.async_copy(
                out_v.at[cur],
                out_hbm.at[pl.ds((base + ch * _CH) * _D, _CH * _D)],
                sem_o[cur])
        for q in range(2):
            if out_cp[q] is not None:
                out_cp[q].wait()
        lane = lax.iota(jnp.int32, 16)
        for k0 in range(_K // 16):
            vals = jnp.zeros((16,), jnp.int32)
            for l in range(16):
                vals = jnp.where(lane == l, hist_s[k0 * 16 + l], vals)
            hist_v[pl.ds(k0 * 16, 16)] = vals
        pltpu.sync_copy(hist_v, hist_hbm.at[pl.ds(wid * _K, _K)])

    return _sc_gather


def _ent_body(h_ref, ent_ref):
    cnt = jnp.sum(h_ref[...].astype(jnp.float32), axis=0, keepdims=True)
    p = cnt * (1.0 / _N)
    ent_ref[0, 0] = -jnp.sum(jnp.where(p > 0, p * jnp.log(p), 0.0)) * (
        1.0 / math.log(2.0))


def _ent_call(hist):
    return pl.pallas_call(
        _ent_body,
        grid=(1,),
        in_specs=[pl.BlockSpec((2 * _NW, _K), lambda i: (0, 0))],
        out_specs=[pl.BlockSpec((1, 1), lambda i: (0, 0),
                                memory_space=pltpu.SMEM)],
        out_shape=[jax.ShapeDtypeStruct((1, 1), jnp.float32)],
    )(hist)


def kernel(input, weight):
    zt = input.transpose(0, 2, 1).reshape(_B * _D, _T)
    w_flat = weight.reshape(_K * _D)
    sc = _make_sc_gather()
    idx0, emb0 = _tc_call(zt, weight, 0)
    x_hat0, hist0 = sc(w_flat, idx0)
    idx1, emb1 = _tc_call(zt, weight, 1)
    x_hat1, hist1 = sc(w_flat, idx1)
    (ent,) = _ent_call(
        jnp.concatenate([hist0, hist1]).reshape(2 * _NW, _K))
    x_hat = jnp.concatenate([x_hat0, x_hat1]).reshape(_B, _T, _D)
    idx = jnp.concatenate([idx0, idx1]).reshape(_B, _T)
    emb_s = emb0[0, 0] + emb1[0, 0]
    return (
        x_hat,
        idx,
        emb_s,
        emb_s,
        ent[0, 0],
    )


# speedup vs baseline: 2.9057x; 1.0002x over previous
import functools
import math

import jax
import jax.numpy as jnp
from jax import lax
from jax.experimental import pallas as pl
from jax.experimental.pallas import tpu as pltpu
from jax.experimental.pallas import tpu_sc as plsc

_K = 512
_D = 32
_B = 128
_T = 1024
_N = _B * _T

_NC = 2
_NS = 16
_NW = _NC * _NS
_HB = _B // 2
_HN = _N // 2
_BPW = _HN // _NW
_CH = 1024


def _tc_body(zt_ref, w_ref, idx_ref, emb_ref):
    b = pl.program_id(0)
    ztb = zt_ref[...]
    w = w_ref[...]
    sqr = jnp.sum(w * w, axis=1, keepdims=True)
    cov2 = jnp.dot(w + w, ztb, preferred_element_type=jnp.float32)
    scores = sqr - cov2
    minv = jnp.min(scores, axis=0)
    iota_c = lax.broadcasted_iota(jnp.int32, (_K, 1), 0).astype(jnp.float32)
    hit = scores <= minv[None, :]
    idx_f = jnp.min(jnp.where(hit, iota_c, float(_K)), axis=0)
    idx_ref[...] = idx_f.astype(jnp.int32)
    part = jnp.sum(minv) + jnp.sum(ztb * ztb)

    @pl.when(b == 0)
    def _init():
        emb_ref[0, 0] = part

    @pl.when(b > 0)
    def _acc():
        emb_ref[0, 0] += part


def _tc_call(zt, w, half):
    off = half * _HB
    return pl.pallas_call(
        _tc_body,
        grid=(_HB,),
        in_specs=[
            pl.BlockSpec((_D, _T), lambda b: (b + off, 0)),
            pl.BlockSpec((_K, _D), lambda b: (0, 0)),
        ],
        out_specs=[
            pl.BlockSpec((_T,), lambda b: (b,)),
            pl.BlockSpec((1, 1), lambda b: (0, 0), memory_space=pltpu.SMEM),
        ],
        out_shape=[
            jax.ShapeDtypeStruct((_HN,), jnp.int32),
            jax.ShapeDtypeStruct((1, 1), jnp.float32),
        ],
    )(zt, w)


@functools.cache
def _make_sc_gather():
    mesh = plsc.VectorSubcoreMesh(core_axis_name="c", subcore_axis_name="s")
    n_chunks = _BPW // _CH

    @functools.partial(
        pl.kernel,
        mesh=mesh,
        out_type=[
            jax.ShapeDtypeStruct((_HN * _D,), jnp.float32),
            jax.ShapeDtypeStruct((_NW * _K,), jnp.int32),
        ],
        scratch_types=[
            pltpu.VMEM((_K * _D,), jnp.float32),
            pltpu.VMEM((2, _CH), jnp.int32),
            pltpu.VMEM((2, _CH * _D), jnp.float32),
            pltpu.SMEM((_K,), jnp.int32),
            pltpu.VMEM((_K,), jnp.int32),
            pltpu.SemaphoreType.DMA,
            pltpu.SemaphoreType.DMA,
            pltpu.SemaphoreType.DMA,
            pltpu.SemaphoreType.DMA,
        ],
    )
    def _sc_gather(w_hbm, idx_hbm, out_hbm, hist_hbm,
                   tab_v, idx_v, out_v, hist_s, hist_v,
                   sem_i0, sem_i1, sem_o0, sem_o1):
        wid = lax.axis_index("s") * _NC + lax.axis_index("c")
        base = wid * _BPW
        sem_i = (sem_i0, sem_i1)
        sem_o = (sem_o0, sem_o1)
        idx_cp = [None, None]
        out_cp = [None, None]
        idx_cp[0] = pltpu.async_copy(
            idx_hbm.at[pl.ds(base, _CH)], idx_v.at[0], sem_i[0])
        pltpu.sync_copy(w_hbm, tab_v)

        def zbody(k0, carry):
            for j in range(8):
                hist_s[k0 * 8 + j] = 0
            return carry

        lax.fori_loop(0, _K // 8, zbody, 0)
        for ch in range(n_chunks):
            cur = ch & 1
            nxt = 1 - cur
            if ch + 1 < n_chunks:
                idx_cp[nxt] = pltpu.async_copy(
                    idx_hbm.at[pl.ds(base + (ch + 1) * _CH, _CH)],
                    idx_v.at[nxt], sem_i[nxt])
            idx_cp[cur].wait()
            if out_cp[cur] is not None:
                out_cp[cur].wait()

            def body(g, carry):
                c_vec = idx_v[cur, pl.ds(g * 16, 16)]
                for l in range(16):
                    c = c_vec[l]
                    b = c * _D
                    t = (g * 16 + l) * _D
                    out_v[cur, pl.ds(t, 16)] = tab_v[pl.ds(b, 16)]
                    out_v[cur, pl.ds(t + 16, 16)] = tab_v[pl.ds(b + 16, 16)]
                    hist_s[c] = hist_s[c] + 1
                return carry

            lax.fori_loop(0, _CH // 16, body, 0)
            out_cp[cur] = pltpu.async_copy(
                out_v.at[cur],
                out_hbm.at[pl.ds((base + ch * _CH) * _D, _CH * _D)],
                sem_o[cur])
        for q in range(2):
            if out_cp[q] is not None:
                out_cp[q].wait()
        lane = lax.iota(jnp.int32, 16)
        for k0 in range(_K // 16):
            vals = jnp.zeros((16,), jnp.int32)
            for l in range(16):
                vals = jnp.where(lane == l, hist_s[k0 * 16 + l], vals)
            hist_v[pl.ds(k0 * 16, 16)] = vals
        pltpu.sync_copy(hist_v, hist_hbm.at[pl.ds(wid * _K, _K)])

    return _sc_gather


def _ent_body(h_ref, ent_ref):
    cnt = jnp.sum(h_ref[...].astype(jnp.float32), axis=0, keepdims=True)
    p = cnt * (1.0 / _N)
    ent_ref[0, 0] = -jnp.sum(jnp.where(p > 0, p * jnp.log(p), 0.0)) * (
        1.0 / math.log(2.0))


def _ent_call(hist):
    return pl.pallas_call(
        _ent_body,
        grid=(1,),
        in_specs=[pl.BlockSpec((2 * _NW, _K), lambda i: (0, 0))],
        out_specs=[pl.BlockSpec((1, 1), lambda i: (0, 0),
                                memory_space=pltpu.SMEM)],
        out_shape=[jax.ShapeDtypeStruct((1, 1), jnp.float32)],
    )(hist)


def kernel(input, weight):
    zt = input.transpose(0, 2, 1).reshape(_B * _D, _T)
    w_flat = weight.reshape(_K * _D)
    sc = _make_sc_gather()
    idx0, emb0 = _tc_call(zt, weight, 0)
    x_hat0, hist0 = sc(w_flat, idx0)
    idx1, emb1 = _tc_call(zt, weight, 1)
    x_hat1, hist1 = sc(w_flat, idx1)
    (ent,) = _ent_call(
        jnp.concatenate([hist0, hist1]).reshape(2 * _NW, _K))
    x_hat = jnp.concatenate([x_hat0, x_hat1]).reshape(_B, _T, _D)
    idx = jnp.concatenate(
        [idx0.reshape(_HB, _T), idx1.reshape(_HB, _T)], axis=0)
    emb_s = emb0[0, 0] + emb1[0, 0]
    return (
        x_hat,
        idx,
        emb_s,
        emb_s,
        ent[0, 0],
    )


# speedup vs baseline: 2.9102x; 1.0016x over previous
import functools
import math

import jax
import jax.numpy as jnp
from jax import lax
from jax.experimental import pallas as pl
from jax.experimental.pallas import tpu as pltpu
from jax.experimental.pallas import tpu_sc as plsc

_K = 512
_D = 32
_B = 128
_T = 1024
_N = _B * _T

_NC = 2
_NS = 16
_NW = _NC * _NS
_HB = _B // 2
_HN = _N // 2
_BPW = _HN // _NW
_CH = 1024


def _tc_body(zt_ref, w_ref, idx_ref, emb_ref):
    b = pl.program_id(0)
    ztb = zt_ref[...]
    w = w_ref[...]
    sqr = jnp.sum(w * w, axis=1, keepdims=True)
    cov2 = jnp.dot(w + w, ztb, preferred_element_type=jnp.float32)
    scores = sqr - cov2
    minv = jnp.min(scores, axis=0)
    iota_c = lax.broadcasted_iota(jnp.int32, (_K, 1), 0).astype(jnp.float32)
    hit = scores <= minv[None, :]
    idx_f = jnp.min(jnp.where(hit, iota_c, float(_K)), axis=0)
    idx_ref[...] = idx_f.astype(jnp.int32)
    part = jnp.sum(minv) + jnp.sum(ztb * ztb)

    @pl.when(b == 0)
    def _init():
        emb_ref[0, 0] = part

    @pl.when(b > 0)
    def _acc():
        emb_ref[0, 0] += part


def _tc_call(zt, w, half):
    off = half * _HB
    return pl.pallas_call(
        _tc_body,
        grid=(_HB,),
        in_specs=[
            pl.BlockSpec((_D, _T), lambda b: (b + off, 0)),
            pl.BlockSpec((_K, _D), lambda b: (0, 0)),
        ],
        out_specs=[
            pl.BlockSpec((_T,), lambda b: (b,)),
            pl.BlockSpec((1, 1), lambda b: (0, 0), memory_space=pltpu.SMEM),
        ],
        out_shape=[
            jax.ShapeDtypeStruct((_HN,), jnp.int32),
            jax.ShapeDtypeStruct((1, 1), jnp.float32),
        ],
    )(zt, w)


@functools.cache
def _make_sc_gather():
    mesh = plsc.VectorSubcoreMesh(core_axis_name="c", subcore_axis_name="s")
    n_chunks = _BPW // _CH

    @functools.partial(
        pl.kernel,
        mesh=mesh,
        out_type=[
            jax.ShapeDtypeStruct((_HN * _D,), jnp.float32),
            jax.ShapeDtypeStruct((_NW * _K,), jnp.int32),
        ],
        scratch_types=[
            pltpu.VMEM((_K * _D,), jnp.float32),
            pltpu.VMEM((2, _CH), jnp.int32),
            pltpu.VMEM((2, _CH * _D), jnp.float32),
            pltpu.SMEM((_K,), jnp.int32),
            pltpu.VMEM((_K,), jnp.int32),
            pltpu.SemaphoreType.DMA,
            pltpu.SemaphoreType.DMA,
            pltpu.SemaphoreType.DMA,
            pltpu.SemaphoreType.DMA,
        ],
    )
    def _sc_gather(w_hbm, idx_hbm, out_hbm, hist_hbm,
                   tab_v, idx_v, out_v, hist_s, hist_v,
                   sem_i0, sem_i1, sem_o0, sem_o1):
        wid = lax.axis_index("s") * _NC + lax.axis_index("c")
        base = wid * _BPW
        sem_i = (sem_i0, sem_i1)
        sem_o = (sem_o0, sem_o1)
        idx_cp = [None, None]
        out_cp = [None, None]
        idx_cp[0] = pltpu.async_copy(
            idx_hbm.at[pl.ds(base, _CH)], idx_v.at[0], sem_i[0])
        pltpu.sync_copy(w_hbm, tab_v)

        def zbody(k0, carry):
            for j in range(8):
                hist_s[k0 * 8 + j] = 0
            return carry

        lax.fori_loop(0, _K // 8, zbody, 0)
        for ch in range(n_chunks):
            cur = ch & 1
            nxt = 1 - cur
            if ch + 1 < n_chunks:
                idx_cp[nxt] = pltpu.async_copy(
                    idx_hbm.at[pl.ds(base + (ch + 1) * _CH, _CH)],
                    idx_v.at[nxt], sem_i[nxt])
            idx_cp[cur].wait()
            if out_cp[cur] is not None:
                out_cp[cur].wait()

            def body(g, carry):
                c_vec = idx_v[cur, pl.ds(g * 16, 16)]
                for l in range(16):
                    c = c_vec[l]
                    b = c * _D
                    t = (g * 16 + l) * _D
                    out_v[cur, pl.ds(t, 16)] = tab_v[pl.ds(b, 16)]
                    out_v[cur, pl.ds(t + 16, 16)] = tab_v[pl.ds(b + 16, 16)]
                    hist_s[c] = hist_s[c] + 1
                return carry

            lax.fori_loop(0, _CH // 16, body, 0)
            out_cp[cur] = pltpu.async_copy(
                out_v.at[cur],
                out_hbm.at[pl.ds((base + ch * _CH) * _D, _CH * _D)],
                sem_o[cur])
        for q in range(2):
            if out_cp[q] is not None:
                out_cp[q].wait()
        lane = lax.iota(jnp.int32, 16)
        for k0 in range(_K // 16):
            vals = jnp.zeros((16,), jnp.int32)
            for l in range(16):
                vals = jnp.where(lane == l, hist_s[k0 * 16 + l], vals)
            hist_v[pl.ds(k0 * 16, 16)] = vals
        pltpu.sync_copy(hist_v, hist_hbm.at[pl.ds(wid * _K, _K)])

    return _sc_gather


def _ent_body(h_ref, ent_ref):
    cnt = jnp.sum(h_ref[...].astype(jnp.float32), axis=0, keepdims=True)
    p = cnt * (1.0 / _N)
    ent_ref[0, 0] = -jnp.sum(jnp.where(p > 0, p * jnp.log(p), 0.0)) * (
        1.0 / math.log(2.0))


def _ent_call(hist):
    return pl.pallas_call(
        _ent_body,
        grid=(1,),
        in_specs=[pl.BlockSpec((2 * _NW, _K), lambda i: (0, 0))],
        out_specs=[pl.BlockSpec((1, 1), lambda i: (0, 0),
                                memory_space=pltpu.SMEM)],
        out_shape=[jax.ShapeDtypeStruct((1, 1), jnp.float32)],
    )(hist)


def kernel(input, weight):
    zt = input.transpose(0, 2, 1).reshape(_B * _D, _T)
    w_flat = weight.reshape(_K * _D)
    sc = _make_sc_gather()
    idx0, emb0 = _tc_call(zt, weight, 0)
    x_hat0, hist0 = sc(w_flat, idx0)
    idx1, emb1 = _tc_call(zt, weight, 1)
    x_hat1, hist1 = sc(w_flat, idx1)
    (ent,) = _ent_call(
        jnp.concatenate([hist0, hist1]).reshape(2 * _NW, _K))
    x_hat = jnp.concatenate(
        [x_hat0.reshape(_HB, _T, _D), x_hat1.reshape(_HB, _T, _D)], axis=0)
    idx = jnp.concatenate(
        [idx0.reshape(_HB, _T), idx1.reshape(_HB, _T)], axis=0)
    emb_s = emb0[0, 0] + emb1[0, 0]
    return (
        x_hat,
        idx,
        emb_s,
        emb_s,
        ent[0, 0],
    )
